# baseline (speedup 1.0000x reference)
import jax
import jax.numpy as jnp
from jax.experimental import pallas as pl
from jax.experimental.pallas import tpu as pltpu

_B = 256
_GRID = 2
_BB = _B // _GRID
_MAXN = 24
_NUM_TYPES = 10
_MAXPOS = 9
_HID = 301
_LAT = 56


def _kern(a_ref, x_ref, p_ref, pos_ref, rcg_ref,
          wih_r_ref, wih_z_ref, wih_n_ref,
          whh_r_ref, whh_z_ref, whh_n_ref,
          bih_r_ref, bih_z_ref, bih_n_ref,
          bhh_r_ref, bhh_z_ref, bhh_n_ref,
          wg_h_ref, wg_p_ref, bg_ref,
          wm_h_ref, wm_p_ref,
          wdf1_ref, bdf1_ref, wdf2_ref, bdf2_ref,
          wfc_h_ref, wfc_f_ref, bfc_ref,
          out_ref):
    f32 = jnp.float32
    wih_r = wih_r_ref[...]
    wih_z = wih_z_ref[...]
    wih_n = wih_n_ref[...]
    whh_r = whh_r_ref[...]
    whh_z = whh_z_ref[...]
    whh_n = whh_n_ref[...]
    bih_r = bih_r_ref[...]
    bih_z = bih_z_ref[...]
    bih_n = bih_n_ref[...]
    bhh_r = bhh_r_ref[...]
    bhh_z = bhh_z_ref[...]
    bhh_n = bhh_n_ref[...]
    wg_h = wg_h_ref[...]
    wg_p = wg_p_ref[...]
    bg = bg_ref[...]
    wm_h = wm_h_ref[...]
    wm_p = wm_p_ref[...]

    grows = []
    hv = None
    for v in range(_MAXN):
        if v == 0:
            hin = jnp.zeros((_BB, _HID), f32)
        else:
            col = a_ref[v]
            terms = [jnp.where(col[:, u:u + 1] < 0.3, grows[u], 0.0)
                     for u in range(v)]
            while len(terms) > 1:
                terms = [terms[i] + terms[i + 1] if i + 1 < len(terms)
                         else terms[i] for i in range(0, len(terms), 2)]
            hin = terms[0]
        xv = x_ref[v]
        r = jax.nn.sigmoid(xv @ wih_r + bih_r + hin @ whh_r + bhh_r)
        z = jax.nn.sigmoid(xv @ wih_z + bih_z + hin @ whh_z + bhh_z)
        n = jnp.tanh(xv @ wih_n + bih_n + r * (hin @ whh_n + bhh_n))
        hv = (1.0 - z) * n + z * hin
        if v < _MAXN - 1:
            pv = p_ref[v]
            gate = jax.nn.sigmoid(hv @ wg_h + pv @ wg_p + bg)
            grows.append(gate * (hv @ wm_h + pv @ wm_p))
    hg = hv

    posq = pos_ref[...]
    j3 = jax.lax.broadcasted_iota(jnp.int32, (_BB, _MAXN, 3 * _MAXPOS), 2)
    pj = j3 // 3
    kj = j3 - pj * 3
    niota = jax.lax.broadcasted_iota(jnp.int32, (_BB, _MAXN, 3 * _MAXPOS), 1) + 1
    m27i = jnp.where(posq[:, :, None] == pj, niota, 0)
    nmax = jnp.max(m27i, axis=1)
    last = jnp.where((m27i == nmax[:, None, :]) & (m27i > 0), 1.0, 0.0)
    r3 = rcg_ref[...]
    rcg27 = (jnp.where(kj == 0, r3[:, :, 0:1], 0.0)
             + jnp.where(kj == 1, r3[:, :, 1:2], 0.0)
             + jnp.where(kj == 2, r3[:, :, 2:3], 0.0))
    df = jnp.sum(last * rcg27, axis=1)

    hdf = jnp.maximum(df @ wdf1_ref[...] + bdf1_ref[...], 0.0)
    hdf = hdf @ wdf2_ref[...] + bdf2_ref[...]

    out_ref[...] = hg @ wfc_h_ref[...] + (0.01 * hdf) @ wfc_f_ref[...] + bfc_ref[...]


def kernel(node_types, node_pos, adj_rand, node_rcg, Wih, Whh, bih, bhh,
           Wg, bg, Wm, Wdf1, bdf1, Wdf2, bdf2, Wfc1, bfc1, Wfc2, bfc2):
    f32 = jnp.float32
    H = _HID
    xt = jax.nn.one_hot(node_types, _NUM_TYPES, dtype=f32)
    xp = jax.nn.one_hot(node_pos, _MAXPOS, dtype=f32)
    x = jnp.concatenate([xt, xp], axis=-1).transpose(1, 0, 2)
    p = xp.transpose(1, 0, 2)
    a = adj_rand.transpose(2, 0, 1)

    args = (
        a, x, p, node_pos.astype(jnp.int32), node_rcg,
        Wih[0:H].T, Wih[H:2 * H].T, Wih[2 * H:].T,
        Whh[0:H].T, Whh[H:2 * H].T, Whh[2 * H:].T,
        bih[0:H][None, :], bih[H:2 * H][None, :], bih[2 * H:][None, :],
        bhh[0:H][None, :], bhh[H:2 * H][None, :], bhh[2 * H:][None, :],
        Wg[:, :H].T, Wg[:, H:].T, bg[None, :],
        Wm[:, :H].T, Wm[:, H:].T,
        Wdf1.T, bdf1[None, :], Wdf2.T, bdf2[None, :],
        jnp.concatenate([Wfc1[:, :H], Wfc2[:, :H]], axis=0).T,
        jnp.concatenate([Wfc1[:, H:], Wfc2[:, H:]], axis=0).T,
        jnp.concatenate([bfc1, bfc2])[None, :],
    )
    def _full(arr):
        nd = arr.ndim
        return pl.BlockSpec(arr.shape, lambda i, _n=nd: (0,) * _n)

    in_specs = [
        pl.BlockSpec((_MAXN, _BB, _MAXN), lambda i: (0, i, 0)),
        pl.BlockSpec((_MAXN, _BB, _NUM_TYPES + _MAXPOS), lambda i: (0, i, 0)),
        pl.BlockSpec((_MAXN, _BB, _MAXPOS), lambda i: (0, i, 0)),
        pl.BlockSpec((_BB, _MAXN), lambda i: (i, 0)),
        pl.BlockSpec((_BB, _MAXN, 3), lambda i: (i, 0, 0)),
    ] + [_full(w) for w in args[5:]]

    return pl.pallas_call(
        _kern,
        grid=(_GRID,),
        in_specs=in_specs,
        out_specs=pl.BlockSpec((_BB, 2 * _LAT), lambda i: (i, 0)),
        out_shape=jax.ShapeDtypeStruct((_B, 2 * _LAT), f32),
        compiler_params=pltpu.CompilerParams(
            dimension_semantics=("parallel",)),
    )(*args)


# speedup vs baseline: 1.0264x; 1.0264x over previous
import jax
import jax.numpy as jnp
from jax.experimental import pallas as pl
from jax.experimental.pallas import tpu as pltpu

_B = 256
_GRID = 1
_BB = _B // _GRID
_MAXN = 24
_NUM_TYPES = 10
_MAXPOS = 9
_HID = 301
_LAT = 56


def _kern(a_ref, x_ref, p_ref, pos_ref, rcg_ref,
          wih_r_ref, wih_z_ref, wih_n_ref,
          whh_r_ref, whh_z_ref, whh_n_ref,
          bih_r_ref, bih_z_ref, bih_n_ref,
          bhh_r_ref, bhh_z_ref, bhh_n_ref,
          wg_h_ref, wg_p_ref, bg_ref,
          wm_h_ref, wm_p_ref,
          wdf1_ref, bdf1_ref, wdf2_ref, bdf2_ref,
          wfc_h_ref, wfc_f_ref, bfc_ref,
          out_ref):
    f32 = jnp.float32
    wih_r = wih_r_ref[...]
    wih_z = wih_z_ref[...]
    wih_n = wih_n_ref[...]
    whh_r = whh_r_ref[...]
    whh_z = whh_z_ref[...]
    whh_n = whh_n_ref[...]
    bih_r = bih_r_ref[...]
    bih_z = bih_z_ref[...]
    bih_n = bih_n_ref[...]
    bhh_r = bhh_r_ref[...]
    bhh_z = bhh_z_ref[...]
    bhh_n = bhh_n_ref[...]
    wg_h = wg_h_ref[...]
    wg_p = wg_p_ref[...]
    bg = bg_ref[...]
    wm_h = wm_h_ref[...]
    wm_p = wm_p_ref[...]

    grows = []
    hv = None
    for v in range(_MAXN):
        if v == 0:
            hin = jnp.zeros((_BB, _HID), f32)
        else:
            col = a_ref[v]
            terms = [jnp.where(col[:, u:u + 1] < 0.3, grows[u], 0.0)
                     for u in range(v)]
            while len(terms) > 1:
                terms = [terms[i] + terms[i + 1] if i + 1 < len(terms)
                         else terms[i] for i in range(0, len(terms), 2)]
            hin = terms[0]
        xv = x_ref[v]
        r = jax.nn.sigmoid(xv @ wih_r + bih_r + hin @ whh_r + bhh_r)
        z = jax.nn.sigmoid(xv @ wih_z + bih_z + hin @ whh_z + bhh_z)
        n = jnp.tanh(xv @ wih_n + bih_n + r * (hin @ whh_n + bhh_n))
        hv = (1.0 - z) * n + z * hin
        if v < _MAXN - 1:
            pv = p_ref[v]
            gate = jax.nn.sigmoid(hv @ wg_h + pv @ wg_p + bg)
            grows.append(gate * (hv @ wm_h + pv @ wm_p))
    hg = hv

    posq = pos_ref[...]
    j3 = jax.lax.broadcasted_iota(jnp.int32, (_BB, _MAXN, 3 * _MAXPOS), 2)
    pj = j3 // 3
    kj = j3 - pj * 3
    niota = jax.lax.broadcasted_iota(jnp.int32, (_BB, _MAXN, 3 * _MAXPOS), 1) + 1
    m27i = jnp.where(posq[:, :, None] == pj, niota, 0)
    nmax = jnp.max(m27i, axis=1)
    last = jnp.where((m27i == nmax[:, None, :]) & (m27i > 0), 1.0, 0.0)
    r3 = rcg_ref[...]
    rcg27 = (jnp.where(kj == 0, r3[:, :, 0:1], 0.0)
             + jnp.where(kj == 1, r3[:, :, 1:2], 0.0)
             + jnp.where(kj == 2, r3[:, :, 2:3], 0.0))
    df = jnp.sum(last * rcg27, axis=1)

    hdf = jnp.maximum(df @ wdf1_ref[...] + bdf1_ref[...], 0.0)
    hdf = hdf @ wdf2_ref[...] + bdf2_ref[...]

    out_ref[...] = hg @ wfc_h_ref[...] + (0.01 * hdf) @ wfc_f_ref[...] + bfc_ref[...]


def kernel(node_types, node_pos, adj_rand, node_rcg, Wih, Whh, bih, bhh,
           Wg, bg, Wm, Wdf1, bdf1, Wdf2, bdf2, Wfc1, bfc1, Wfc2, bfc2):
    f32 = jnp.float32
    H = _HID
    xt = jax.nn.one_hot(node_types, _NUM_TYPES, dtype=f32)
    xp = jax.nn.one_hot(node_pos, _MAXPOS, dtype=f32)
    x = jnp.concatenate([xt, xp], axis=-1).transpose(1, 0, 2)
    p = xp.transpose(1, 0, 2)
    a = adj_rand.transpose(2, 0, 1)

    args = (
        a, x, p, node_pos.astype(jnp.int32), node_rcg,
        Wih[0:H].T, Wih[H:2 * H].T, Wih[2 * H:].T,
        Whh[0:H].T, Whh[H:2 * H].T, Whh[2 * H:].T,
        bih[0:H][None, :], bih[H:2 * H][None, :], bih[2 * H:][None, :],
        bhh[0:H][None, :], bhh[H:2 * H][None, :], bhh[2 * H:][None, :],
        Wg[:, :H].T, Wg[:, H:].T, bg[None, :],
        Wm[:, :H].T, Wm[:, H:].T,
        Wdf1.T, bdf1[None, :], Wdf2.T, bdf2[None, :],
        jnp.concatenate([Wfc1[:, :H], Wfc2[:, :H]], axis=0).T,
        jnp.concatenate([Wfc1[:, H:], Wfc2[:, H:]], axis=0).T,
        jnp.concatenate([bfc1, bfc2])[None, :],
    )
    def _full(arr):
        nd = arr.ndim
        return pl.BlockSpec(arr.shape, lambda i, _n=nd: (0,) * _n)

    in_specs = [
        pl.BlockSpec((_MAXN, _BB, _MAXN), lambda i: (0, i, 0)),
        pl.BlockSpec((_MAXN, _BB, _NUM_TYPES + _MAXPOS), lambda i: (0, i, 0)),
        pl.BlockSpec((_MAXN, _BB, _MAXPOS), lambda i: (0, i, 0)),
        pl.BlockSpec((_BB, _MAXN), lambda i: (i, 0)),
        pl.BlockSpec((_BB, _MAXN, 3), lambda i: (i, 0, 0)),
    ] + [_full(w) for w in args[5:]]

    return pl.pallas_call(
        _kern,
        grid=(_GRID,),
        in_specs=in_specs,
        out_specs=pl.BlockSpec((_BB, 2 * _LAT), lambda i: (i, 0)),
        out_shape=jax.ShapeDtypeStruct((_B, 2 * _LAT), f32),
        compiler_params=pltpu.CompilerParams(
            dimension_semantics=("parallel",)),
    )(*args)


# speedup vs baseline: 1.0880x; 1.0600x over previous
import jax
import jax.numpy as jnp
from jax.experimental import pallas as pl

_B = 256
_MAXN = 24
_NUM_TYPES = 10
_MAXPOS = 9
_HID = 301
_LAT = 56


def _mm_t(x, w):
    return jax.lax.dot_general(x, w, (((1,), (1,)), ((), ())),
                               preferred_element_type=jnp.float32)


def _kern(a_ref, x_ref, p_ref, pos_ref, rcg_ref,
          wih_r_ref, wih_z_ref, wih_n_ref,
          whh_r_ref, whh_z_ref, whh_n_ref,
          bih_r_ref, bih_z_ref, bih_n_ref,
          bhh_r_ref, bhh_z_ref, bhh_n_ref,
          wg_h_ref, wg_p_ref, bg_ref,
          wm_h_ref, wm_p_ref,
          wdf1_ref, bdf1_ref, wdf2_ref, bdf2_ref,
          wfc_h_ref, wfc_f_ref, bfc_ref,
          out_ref):
    f32 = jnp.float32
    wih_r = wih_r_ref[...]
    wih_z = wih_z_ref[...]
    wih_n = wih_n_ref[...]
    whh_r = whh_r_ref[...]
    whh_z = whh_z_ref[...]
    whh_n = whh_n_ref[...]
    bih_r = bih_r_ref[...]
    bih_z = bih_z_ref[...]
    bih_n = bih_n_ref[...]
    bhh_r = bhh_r_ref[...]
    bhh_z = bhh_z_ref[...]
    bhh_n = bhh_n_ref[...]
    wg_h = wg_h_ref[...]
    wg_p = wg_p_ref[...]
    bg = bg_ref[...]
    wm_h = wm_h_ref[...]
    wm_p = wm_p_ref[...]

    grows = []
    hv = None
    for v in range(_MAXN):
        if v == 0:
            hin = jnp.zeros((_B, _HID), f32)
        else:
            col = a_ref[v]
            terms = [jnp.where(col[:, u:u + 1] < 0.3, grows[u], 0.0)
                     for u in range(v)]
            while len(terms) > 1:
                terms = [terms[i] + terms[i + 1] if i + 1 < len(terms)
                         else terms[i] for i in range(0, len(terms), 2)]
            hin = terms[0]
        xv = x_ref[v]
        r = jax.nn.sigmoid(_mm_t(xv, wih_r) + bih_r + _mm_t(hin, whh_r) + bhh_r)
        z = jax.nn.sigmoid(_mm_t(xv, wih_z) + bih_z + _mm_t(hin, whh_z) + bhh_z)
        n = jnp.tanh(_mm_t(xv, wih_n) + bih_n + r * (_mm_t(hin, whh_n) + bhh_n))
        hv = (1.0 - z) * n + z * hin
        if v < _MAXN - 1:
            pv = p_ref[v]
            gate = jax.nn.sigmoid(_mm_t(hv, wg_h) + _mm_t(pv, wg_p) + bg)
            grows.append(gate * (_mm_t(hv, wm_h) + _mm_t(pv, wm_p)))
    hg = hv

    posq = pos_ref[...]
    j3 = jax.lax.broadcasted_iota(jnp.int32, (_B, _MAXN, 3 * _MAXPOS), 2)
    pj = j3 // 3
    kj = j3 - pj * 3
    niota = jax.lax.broadcasted_iota(jnp.int32, (_B, _MAXN, 3 * _MAXPOS), 1) + 1
    m27i = jnp.where(posq[:, :, None] == pj, niota, 0)
    nmax = jnp.max(m27i, axis=1)
    last = jnp.where((m27i == nmax[:, None, :]) & (m27i > 0), 1.0, 0.0)
    r3 = rcg_ref[...]
    rcg27 = (jnp.where(kj == 0, r3[:, :, 0:1], 0.0)
             + jnp.where(kj == 1, r3[:, :, 1:2], 0.0)
             + jnp.where(kj == 2, r3[:, :, 2:3], 0.0))
    df = jnp.sum(last * rcg27, axis=1)

    hdf = jnp.maximum(_mm_t(df, wdf1_ref[...]) + bdf1_ref[...], 0.0)
    hdf = _mm_t(hdf, wdf2_ref[...]) + bdf2_ref[...]

    out_ref[...] = (_mm_t(hg, wfc_h_ref[...])
                    + _mm_t(0.01 * hdf, wfc_f_ref[...]) + bfc_ref[...])


def kernel(node_types, node_pos, adj_rand, node_rcg, Wih, Whh, bih, bhh,
           Wg, bg, Wm, Wdf1, bdf1, Wdf2, bdf2, Wfc1, bfc1, Wfc2, bfc2):
    f32 = jnp.float32
    H = _HID
    tt = node_types.T
    pt = node_pos.T
    x = jnp.concatenate([jax.nn.one_hot(tt, _NUM_TYPES, dtype=f32),
                         jax.nn.one_hot(pt, _MAXPOS, dtype=f32)], axis=-1)
    p = jax.nn.one_hot(pt, _MAXPOS, dtype=f32)
    a = adj_rand.transpose(2, 0, 1)

    args = (
        a, x, p, node_pos.astype(jnp.int32), node_rcg,
        Wih[0:H], Wih[H:2 * H], Wih[2 * H:],
        Whh[0:H], Whh[H:2 * H], Whh[2 * H:],
        bih[0:H][None, :], bih[H:2 * H][None, :], bih[2 * H:][None, :],
        bhh[0:H][None, :], bhh[H:2 * H][None, :], bhh[2 * H:][None, :],
        Wg[:, :H], Wg[:, H:], bg[None, :],
        Wm[:, :H], Wm[:, H:],
        Wdf1, bdf1[None, :], Wdf2, bdf2[None, :],
        jnp.concatenate([Wfc1[:, :H], Wfc2[:, :H]], axis=0),
        jnp.concatenate([Wfc1[:, H:], Wfc2[:, H:]], axis=0),
        jnp.concatenate([bfc1, bfc2])[None, :],
    )
    return pl.pallas_call(
        _kern,
        out_shape=jax.ShapeDtypeStruct((_B, 2 * _LAT), f32),
    )(*args)


# speedup vs baseline: 1.1371x; 1.0451x over previous
import jax
import jax.numpy as jnp
from jax.experimental import pallas as pl

_B = 256
_MAXN = 24
_NUM_TYPES = 10
_MAXPOS = 9
_HID = 301
_LAT = 56


def _mm_t(x, w):
    return jax.lax.dot_general(x, w, (((1,), (1,)), ((), ())),
                               preferred_element_type=jnp.float32)


def _kern(a_ref, types_ref, pos_ref, rcg_ref,
          wih_t_r_ref, wih_t_z_ref, wih_t_n_ref,
          wih_p_r_ref, wih_p_z_ref, wih_p_n_ref,
          whh_r_ref, whh_z_ref, whh_n_ref,
          b6_ref,
          wg_h_ref, wg_p_ref, bg_ref,
          wm_h_ref, wm_p_ref,
          wdf1_ref, bdf1_ref, wdf2_ref, bdf2_ref,
          wfc_h_ref, wfc_f_ref, bfc_ref,
          out_ref):
    f32 = jnp.float32
    wih_t_r = wih_t_r_ref[...]
    wih_t_z = wih_t_z_ref[...]
    wih_t_n = wih_t_n_ref[...]
    wih_p_r = wih_p_r_ref[...]
    wih_p_z = wih_p_z_ref[...]
    wih_p_n = wih_p_n_ref[...]
    whh_r = whh_r_ref[...]
    whh_z = whh_z_ref[...]
    whh_n = whh_n_ref[...]
    b6 = b6_ref[...]
    bih_r, bih_z, bih_n = b6[0:1], b6[1:2], b6[2:3]
    bhh_r, bhh_z, bhh_n = b6[3:4], b6[4:5], b6[5:6]
    wg_h = wg_h_ref[...]
    wg_p = wg_p_ref[...]
    bg = bg_ref[...]
    wm_h = wm_h_ref[...]
    wm_p = wm_p_ref[...]

    types = types_ref[...]
    posq = pos_ref[...]
    iota_t = jax.lax.broadcasted_iota(jnp.int32, (_B, _NUM_TYPES), 1)
    iota_p = jax.lax.broadcasted_iota(jnp.int32, (_B, _MAXPOS), 1)

    grows = []
    hv = None
    for v in range(_MAXN):
        if v == 0:
            hin = jnp.zeros((_B, _HID), f32)
        else:
            terms = [jnp.where(a_ref[:, 24 * v + u:24 * v + u + 1] < 0.3,
                               grows[u], 0.0)
                     for u in range(v)]
            while len(terms) > 1:
                terms = [terms[i] + terms[i + 1] if i + 1 < len(terms)
                         else terms[i] for i in range(0, len(terms), 2)]
            hin = terms[0]
        oh_t = jnp.where(types[:, v:v + 1] == iota_t, 1.0, 0.0)
        oh_p = jnp.where(posq[:, v:v + 1] == iota_p, 1.0, 0.0)
        r = jax.nn.sigmoid(_mm_t(oh_t, wih_t_r) + _mm_t(oh_p, wih_p_r)
                           + bih_r + _mm_t(hin, whh_r) + bhh_r)
        z = jax.nn.sigmoid(_mm_t(oh_t, wih_t_z) + _mm_t(oh_p, wih_p_z)
                           + bih_z + _mm_t(hin, whh_z) + bhh_z)
        n = jnp.tanh(_mm_t(oh_t, wih_t_n) + _mm_t(oh_p, wih_p_n)
                     + bih_n + r * (_mm_t(hin, whh_n) + bhh_n))
        hv = (1.0 - z) * n + z * hin
        if v < _MAXN - 1:
            gate = jax.nn.sigmoid(_mm_t(hv, wg_h) + _mm_t(oh_p, wg_p) + bg)
            grows.append(gate * (_mm_t(hv, wm_h) + _mm_t(oh_p, wm_p)))
    hg = hv

    j3 = jax.lax.broadcasted_iota(jnp.int32, (_B, _MAXN, 3 * _MAXPOS), 2)
    pj = j3 // 3
    kj = j3 - pj * 3
    niota = jax.lax.broadcasted_iota(jnp.int32, (_B, _MAXN, 3 * _MAXPOS), 1) + 1
    m27i = jnp.where(posq[:, :, None] == pj, niota, 0)
    nmax = jnp.max(m27i, axis=1)
    last = jnp.where((m27i == nmax[:, None, :]) & (m27i > 0), 1.0, 0.0)
    r3 = rcg_ref[...]
    rcg27 = (jnp.where(kj == 0, r3[:, :, 0:1], 0.0)
             + jnp.where(kj == 1, r3[:, :, 1:2], 0.0)
             + jnp.where(kj == 2, r3[:, :, 2:3], 0.0))
    df = jnp.sum(last * rcg27, axis=1)

    hdf = jnp.maximum(_mm_t(df, wdf1_ref[...]) + bdf1_ref[...], 0.0)
    hdf = _mm_t(hdf, wdf2_ref[...]) + bdf2_ref[...]

    out_ref[...] = (_mm_t(hg, wfc_h_ref[...])
                    + _mm_t(0.01 * hdf, wfc_f_ref[...]) + bfc_ref[...])


def kernel(node_types, node_pos, adj_rand, node_rcg, Wih, Whh, bih, bhh,
           Wg, bg, Wm, Wdf1, bdf1, Wdf2, bdf2, Wfc1, bfc1, Wfc2, bfc2):
    H = _HID
    NT = _NUM_TYPES
    a = adj_rand.transpose(0, 2, 1).reshape(_B, _MAXN * _MAXN)

    args = (
        a, node_types.astype(jnp.int32), node_pos.astype(jnp.int32), node_rcg,
        Wih[0:H, :NT], Wih[H:2 * H, :NT], Wih[2 * H:, :NT],
        Wih[0:H, NT:], Wih[H:2 * H, NT:], Wih[2 * H:, NT:],
        Whh[0:H], Whh[H:2 * H], Whh[2 * H:],
        jnp.concatenate([bih.reshape(3, H), bhh.reshape(3, H)], axis=0),
        Wg[:, :H], Wg[:, H:], bg[None, :],
        Wm[:, :H], Wm[:, H:],
        Wdf1, bdf1[None, :], Wdf2, bdf2[None, :],
        jnp.concatenate([Wfc1[:, :H], Wfc2[:, :H]], axis=0),
        jnp.concatenate([Wfc1[:, H:], Wfc2[:, H:]], axis=0),
        jnp.concatenate([bfc1, bfc2])[None, :],
    )
    return pl.pallas_call(
        _kern,
        out_shape=jax.ShapeDtypeStruct((_B, 2 * _LAT), jnp.float32),
    )(*args)
